# Initial kernel scaffold; baseline (speedup 1.0000x reference)
import jax
import jax.numpy as jnp
from jax.experimental import pallas as pl


def kernel(src_features, edge_sh, edge_scalars, edge_index, W1, b1, W2, b2, bn_weight, bn_bias):
    raise NotImplementedError("write your pallas kernel here")


# speedup vs baseline: 5.6538x; 5.6538x over previous
import functools

import jax
import jax.numpy as jnp
import numpy as np
from jax import lax
from jax.experimental import pallas as pl
from jax.experimental.pallas import tpu as pltpu
from jax.experimental.pallas import tpu_sc as plsc

N_NODES = 10000
N_EDGES = 320000
MUL = 8
IN_DIM = 32
SH_DIM = 4
SCAL_DIM = 32
HID = 32
EPS = 1e-5
ALPHA = 0.25
INV_SQRT3 = 1.0 / np.sqrt(3.0)

CHUNK = 128
R_GROUPS = N_EDGES // CHUNK
NC, NS = 2, 16
NW = NC * NS
ROWS_PER_W = -(-R_GROUPS // NW)

BLK = 512
N_BLKS = N_EDGES // BLK

OUT_W = 48
NODES_PER_SUB = N_NODES // NS


def _build_constants():
    ebig = np.zeros((64, 512), np.float32)
    f = np.zeros((256, 512), np.float32)
    rbig = np.zeros((512, 56), np.float32)
    for u in range(MUL):
        for w in range(MUL):
            ebig[u, 8 * u + w] = 1.0
            ebig[8 + u, 64 + 8 * u + w] = 1.0
            f[8 * u + w, 8 * u + w] = 1.0
            f[64 + 8 * u + w, 64 + 8 * u + w] = 1.0
            rbig[8 * u + w, w] = 1.0
            for i in range(3):
                ebig[16 + 3 * u + i, 128 + 24 * u + 3 * w + i] = 1.0
                ebig[40 + 3 * u + i, 320 + 24 * u + 3 * w + i] = 1.0
                f[128 + 8 * u + w, 128 + 24 * u + 3 * w + i] = 1.0
                f[192 + 8 * u + w, 320 + 24 * u + 3 * w + i] = 1.0
                rbig[320 + 24 * u + 3 * w + i, w] = 1.0
                rbig[64 + 8 * u + w, 8 + 3 * w + i] = 1.0
                rbig[128 + 24 * u + 3 * w + i, 32 + 3 * w + i] = 1.0
    pre = np.zeros((SH_DIM, 24), np.float32)
    for k in range(MUL):
        for i in range(3):
            pre[1 + i, 3 * k + i] = 1.0
    sv3 = np.zeros((24, 8), np.float32)
    r824 = np.zeros((8, 24), np.float32)
    for u in range(MUL):
        for i in range(3):
            sv3[3 * u + i, u] = 1.0 / 3.0
            r824[u, 3 * u + i] = 1.0
    return ebig, f, rbig, pre, sv3, r824


_EBIG_NP, _F_NP, _RBIG_NP, _PRE_NP, _SV3_NP, _R824_NP = _build_constants()


@functools.cache
def _make_gather_kernel():
    mesh = plsc.VectorSubcoreMesh(
        core_axis_name="c", subcore_axis_name="s", num_cores=NC,
        num_subcores=NS)

    @functools.partial(
        pl.kernel,
        out_type=jax.ShapeDtypeStruct((N_EDGES, IN_DIM), jnp.float32),
        mesh=mesh,
        compiler_params=pltpu.CompilerParams(use_tc_tiling_on_sc=False),
        scratch_types=[
            pltpu.VMEM((CHUNK,), jnp.int32),
            pltpu.VMEM((CHUNK, IN_DIM), jnp.float32),
            pltpu.SemaphoreType.DMA,
        ],
    )
    def _gather_kernel(src_hbm, idx_hbm, out_hbm, idx_v, rows_v, sem):
        wid = lax.axis_index("s") * NC + lax.axis_index("c")

        def body(j, carry):
            r = wid + j * NW

            @pl.when(r < R_GROUPS)
            def _():
                pltpu.sync_copy(idx_hbm.at[r], idx_v)
                pltpu.async_copy(src_hbm.at[idx_v], rows_v, sem).wait()
                pltpu.sync_copy(rows_v, out_hbm.at[pl.ds(r * CHUNK, CHUNK)])

            return carry

        lax.fori_loop(0, ROWS_PER_W, body, 0)

    return _gather_kernel


def _edge_body(es_ref, xg_ref, sh_ref, win_ref, bin_ref, w2f_ref, b2f_ref,
               ebig_ref, rbig_ref, out_ref):
    es = es_ref[...]
    sh = sh_ref[...]
    x = xg_ref[...]
    cat = jnp.concatenate([es, sh], axis=1)
    t0 = jnp.dot(cat.astype(jnp.bfloat16), win_ref[...],
                 preferred_element_type=jnp.float32) + bin_ref[...]
    pre = t0[:, :HID]
    shv_t = t0[:, HID:HID + 24]
    h = pre * 0.5 * (1.0 + lax.erf(pre * np.float32(1.0 / np.sqrt(2.0))))
    wexp = jnp.dot(h.astype(jnp.bfloat16), w2f_ref[...],
                   preferred_element_type=jnp.float32) + b2f_ref[...]
    xs = x[:, :MUL]
    xv = x[:, MUL:]
    shs = sh[:, 0:1]
    ain = jnp.concatenate(
        [xs * shs, xs, xv * shs, np.float32(INV_SQRT3) * xv * shv_t], axis=1)
    aexp = jnp.dot(ain.astype(jnp.bfloat16), ebig_ref[...],
                   preferred_element_type=jnp.float32)
    p = aexp * wexp
    o = jnp.dot(p.astype(jnp.bfloat16), rbig_ref[...],
                preferred_element_type=jnp.float32)
    out_s = np.float32(ALPHA) * o[:, 0:8]
    out_v = np.float32(ALPHA) * (o[:, 8:32] * shv_t + o[:, 32:56])
    ones = jnp.ones((BLK, 1), jnp.float32)
    zeros = jnp.zeros((BLK, OUT_W - 33), jnp.float32)
    out_ref[...] = jnp.concatenate([out_s, out_v, ones, zeros], axis=1)


_edge_call = pl.pallas_call(
    _edge_body,
    grid=(N_BLKS,),
    in_specs=[
        pl.BlockSpec((BLK, SCAL_DIM), lambda i: (i, 0)),
        pl.BlockSpec((BLK, IN_DIM), lambda i: (i, 0)),
        pl.BlockSpec((BLK, SH_DIM), lambda i: (i, 0)),
        pl.BlockSpec((36, 56), lambda i: (0, 0)),
        pl.BlockSpec((1, 56), lambda i: (0, 0)),
        pl.BlockSpec((HID, 512), lambda i: (0, 0)),
        pl.BlockSpec((1, 512), lambda i: (0, 0)),
        pl.BlockSpec((64, 512), lambda i: (0, 0)),
        pl.BlockSpec((512, 56), lambda i: (0, 0)),
    ],
    out_specs=pl.BlockSpec((BLK, OUT_W), lambda i: (i, 0)),
    out_shape=jax.ShapeDtypeStruct((N_EDGES, OUT_W), jnp.float32),
)


@functools.cache
def _make_scatter_kernel():
    mesh = plsc.VectorSubcoreMesh(
        core_axis_name="c", subcore_axis_name="s", num_cores=NC,
        num_subcores=NS)

    @functools.partial(
        pl.kernel,
        out_type=jax.ShapeDtypeStruct((NC, N_NODES, OUT_W), jnp.float32),
        mesh=mesh,
        compiler_params=pltpu.CompilerParams(use_tc_tiling_on_sc=False),
        scratch_types=[
            pltpu.VMEM((CHUNK,), jnp.int32),
            pltpu.VMEM((CHUNK, OUT_W), jnp.float32),
            pltpu.VMEM_SHARED((N_NODES, OUT_W), jnp.float32),
        ],
    )
    def _scatter_kernel(rows_hbm, idx_hbm, zeros_hbm, out_hbm, idx_v, rows_v,
                        acc):
        c = lax.axis_index("c")
        s = lax.axis_index("s")
        wid = s * NC + c
        sl = pl.ds(s * NODES_PER_SUB, NODES_PER_SUB)
        pltpu.sync_copy(zeros_hbm.at[sl], acc.at[sl])
        plsc.subcore_barrier()

        def body(j, carry):
            r = wid + j * NW

            @pl.when(r < R_GROUPS)
            def _():
                pltpu.sync_copy(idx_hbm.at[r], idx_v)
                pltpu.sync_copy(rows_hbm.at[pl.ds(r * CHUNK, CHUNK)], rows_v)
                pltpu.sync_copy(rows_v, acc.at[idx_v], add=True)

            return carry

        lax.fori_loop(0, ROWS_PER_W, body, 0)
        plsc.subcore_barrier()
        pltpu.sync_copy(acc.at[sl], out_hbm.at[c, sl])

    return _scatter_kernel


def _final_body(p_ref, bnw_ref, bnb_ref, sv3_ref, r824_ref, out_ref):
    p = p_ref[...]
    sums = p[0] + p[1]
    cnt = jnp.maximum(sums[:, 32:33], 1.0)
    o = sums[:, :IN_DIM] / cnt
    s = o[:, :MUL]
    v = o[:, MUL:]
    s_mean = jnp.mean(s, axis=0, keepdims=True)
    s_c = s - s_mean
    s_var = jnp.mean(s_c * s_c, axis=0, keepdims=True)
    bnw = bnw_ref[...]
    s_out = s_c * (lax.rsqrt(s_var + EPS) * bnw[:, :MUL]) + bnb_ref[...]
    vsq = jnp.mean(v * v, axis=0, keepdims=True)
    v_norm = jnp.dot(vsq, sv3_ref[...], preferred_element_type=jnp.float32)
    vfac = lax.rsqrt(v_norm + EPS) * bnw[:, MUL:]
    vfac24 = jnp.dot(vfac, r824_ref[...], preferred_element_type=jnp.float32)
    out_ref[...] = jnp.concatenate([s_out, v * vfac24], axis=1)


_final_call = pl.pallas_call(
    _final_body,
    in_specs=[
        pl.BlockSpec((NC, N_NODES, OUT_W), lambda: (0, 0, 0)),
        pl.BlockSpec((1, 2 * MUL), lambda: (0, 0)),
        pl.BlockSpec((1, MUL), lambda: (0, 0)),
        pl.BlockSpec((24, 8), lambda: (0, 0)),
        pl.BlockSpec((8, 24), lambda: (0, 0)),
    ],
    out_specs=pl.BlockSpec((N_NODES, IN_DIM), lambda: (0, 0)),
    out_shape=jax.ShapeDtypeStruct((N_NODES, IN_DIM), jnp.float32),
)


def kernel(src_features, edge_sh, edge_scalars, edge_index, W1, b1, W2, b2,
           bn_weight, bn_bias):
    src2d = edge_index[0].reshape(R_GROUPS, CHUNK)
    dst2d = edge_index[1].reshape(R_GROUPS, CHUNK)
    win = (
        jnp.zeros((36, 56), jnp.float32)
        .at[:SCAL_DIM, :HID].set(W1)
        .at[SCAL_DIM:, HID:].set(_PRE_NP)
        .astype(jnp.bfloat16)
    )
    b_in = jnp.concatenate([b1, jnp.zeros((24,), jnp.float32)]).reshape(1, 56)
    w2f = (W2 @ _F_NP).astype(jnp.bfloat16)
    b2f = (b2 @ _F_NP).reshape(1, 512)
    x_g = _make_gather_kernel()(src_features, src2d)
    out_tp = _edge_call(edge_scalars, x_g, edge_sh, win, b_in, w2f, b2f,
                        _EBIG_NP.astype(jnp.bfloat16),
                        _RBIG_NP.astype(jnp.bfloat16))
    partial = _make_scatter_kernel()(out_tp, dst2d,
                                     jnp.zeros((N_NODES, OUT_W), jnp.float32))
    return _final_call(partial, bn_weight.reshape(1, 2 * MUL),
                       bn_bias.reshape(1, MUL), _SV3_NP, _R824_NP)


# speedup vs baseline: 6.9722x; 1.2332x over previous
import functools

import jax
import jax.numpy as jnp
import numpy as np
from jax import lax
from jax.experimental import pallas as pl
from jax.experimental.pallas import tpu as pltpu
from jax.experimental.pallas import tpu_sc as plsc

N_NODES = 10000
N_EDGES = 320000
MUL = 8
IN_DIM = 32
SH_DIM = 4
SCAL_DIM = 32
HID = 32
EPS = 1e-5
ALPHA = 0.25
INV_SQRT3 = 1.0 / np.sqrt(3.0)

CHUNK = 128
R_GROUPS = N_EDGES // CHUNK
NC, NS = 2, 16
NW = NC * NS
ROWS_PER_W = -(-R_GROUPS // NW)

BLK = 1280
N_BLKS = N_EDGES // BLK

OUT_W = 48
NODES_PER_SUB = N_NODES // NS


def _build_constants():
    ebig = np.zeros((64, 512), np.float32)
    f = np.zeros((256, 512), np.float32)
    rbig = np.zeros((512, 56), np.float32)
    for u in range(MUL):
        for w in range(MUL):
            ebig[u, 8 * u + w] = 1.0
            ebig[8 + u, 64 + 8 * u + w] = 1.0
            f[8 * u + w, 8 * u + w] = 1.0
            f[64 + 8 * u + w, 64 + 8 * u + w] = 1.0
            rbig[8 * u + w, w] = 1.0
            for i in range(3):
                ebig[16 + 3 * u + i, 128 + 24 * u + 3 * w + i] = 1.0
                ebig[40 + 3 * u + i, 320 + 24 * u + 3 * w + i] = 1.0
                f[128 + 8 * u + w, 128 + 24 * u + 3 * w + i] = 1.0
                f[192 + 8 * u + w, 320 + 24 * u + 3 * w + i] = 1.0
                rbig[320 + 24 * u + 3 * w + i, w] = 1.0
                rbig[64 + 8 * u + w, 8 + 3 * w + i] = 1.0
                rbig[128 + 24 * u + 3 * w + i, 32 + 3 * w + i] = 1.0
    pre = np.zeros((SH_DIM, 24), np.float32)
    for k in range(MUL):
        for i in range(3):
            pre[1 + i, 3 * k + i] = 1.0
    sv3 = np.zeros((24, 8), np.float32)
    r824 = np.zeros((8, 24), np.float32)
    for u in range(MUL):
        for i in range(3):
            sv3[3 * u + i, u] = 1.0 / 3.0
            r824[u, 3 * u + i] = 1.0
    return ebig, f, rbig, pre, sv3, r824


_EBIG_NP, _F_NP, _RBIG_NP, _PRE_NP, _SV3_NP, _R824_NP = _build_constants()


@functools.cache
def _make_gather_kernel():
    mesh = plsc.VectorSubcoreMesh(
        core_axis_name="c", subcore_axis_name="s", num_cores=NC,
        num_subcores=NS)

    @functools.partial(
        pl.kernel,
        out_type=jax.ShapeDtypeStruct((N_EDGES, IN_DIM), jnp.float32),
        mesh=mesh,
        compiler_params=pltpu.CompilerParams(use_tc_tiling_on_sc=False),
        scratch_types=[
            pltpu.VMEM((CHUNK,), jnp.int32),
            pltpu.VMEM((CHUNK, IN_DIM), jnp.float32),
            pltpu.SemaphoreType.DMA,
        ],
    )
    def _gather_kernel(src_hbm, idx_hbm, out_hbm, idx_v, rows_v, sem):
        wid = lax.axis_index("s") * NC + lax.axis_index("c")

        def body(j, carry):
            r = wid + j * NW

            @pl.when(r < R_GROUPS)
            def _():
                pltpu.sync_copy(idx_hbm.at[r], idx_v)
                pltpu.async_copy(src_hbm.at[idx_v], rows_v, sem).wait()
                pltpu.sync_copy(rows_v, out_hbm.at[pl.ds(r * CHUNK, CHUNK)])

            return carry

        lax.fori_loop(0, ROWS_PER_W, body, 0)

    return _gather_kernel


def _edge_body(es_ref, xg_ref, sh_ref, win_ref, bin_ref, w2f_ref, b2f_ref,
               ebig_ref, rbig_ref, out_ref):
    es = es_ref[...]
    sh = sh_ref[...]
    x = xg_ref[...]
    cat = jnp.concatenate([es, sh], axis=1)
    t0 = jnp.dot(cat.astype(jnp.bfloat16), win_ref[...],
                 preferred_element_type=jnp.float32) + bin_ref[...]
    pre = t0[:, :HID]
    shv_t = t0[:, HID:HID + 24]
    h = pre * 0.5 * (1.0 + lax.erf(pre * np.float32(1.0 / np.sqrt(2.0))))
    wexp = jnp.dot(h.astype(jnp.bfloat16), w2f_ref[...],
                   preferred_element_type=jnp.float32).astype(jnp.bfloat16)
    wexp = wexp + b2f_ref[...]
    xs = x[:, :MUL]
    xv = x[:, MUL:]
    shs = sh[:, 0:1]
    ain = jnp.concatenate(
        [xs * shs, xs, xv * shs, np.float32(INV_SQRT3) * xv * shv_t], axis=1)
    aexp = jnp.dot(ain.astype(jnp.bfloat16), ebig_ref[...],
                   preferred_element_type=jnp.float32).astype(jnp.bfloat16)
    p = aexp * wexp
    o = jnp.dot(p, rbig_ref[...],
                preferred_element_type=jnp.float32)
    out_s = np.float32(ALPHA) * o[:, 0:8]
    out_v = np.float32(ALPHA) * (o[:, 8:32] * shv_t + o[:, 32:56])
    ones = jnp.ones((BLK, 1), jnp.float32)
    zeros = jnp.zeros((BLK, OUT_W - 33), jnp.float32)
    out_ref[...] = jnp.concatenate([out_s, out_v, ones, zeros], axis=1)


_edge_call = pl.pallas_call(
    _edge_body,
    grid=(N_BLKS,),
    in_specs=[
        pl.BlockSpec((BLK, SCAL_DIM), lambda i: (i, 0)),
        pl.BlockSpec((BLK, IN_DIM), lambda i: (i, 0)),
        pl.BlockSpec((BLK, SH_DIM), lambda i: (i, 0)),
        pl.BlockSpec((36, 56), lambda i: (0, 0)),
        pl.BlockSpec((1, 56), lambda i: (0, 0)),
        pl.BlockSpec((HID, 512), lambda i: (0, 0)),
        pl.BlockSpec((1, 512), lambda i: (0, 0)),
        pl.BlockSpec((64, 512), lambda i: (0, 0)),
        pl.BlockSpec((512, 56), lambda i: (0, 0)),
    ],
    out_specs=pl.BlockSpec((BLK, OUT_W), lambda i: (i, 0)),
    out_shape=jax.ShapeDtypeStruct((N_EDGES, OUT_W), jnp.float32),
)


@functools.cache
def _make_scatter_kernel():
    mesh = plsc.VectorSubcoreMesh(
        core_axis_name="c", subcore_axis_name="s", num_cores=NC,
        num_subcores=NS)

    @functools.partial(
        pl.kernel,
        out_type=jax.ShapeDtypeStruct((NC, N_NODES, OUT_W), jnp.float32),
        mesh=mesh,
        compiler_params=pltpu.CompilerParams(use_tc_tiling_on_sc=False),
        scratch_types=[
            pltpu.VMEM((CHUNK,), jnp.int32),
            pltpu.VMEM((CHUNK, OUT_W), jnp.float32),
            pltpu.VMEM_SHARED((N_NODES, OUT_W), jnp.float32),
        ],
    )
    def _scatter_kernel(rows_hbm, idx_hbm, zeros_hbm, out_hbm, idx_v, rows_v,
                        acc):
        c = lax.axis_index("c")
        s = lax.axis_index("s")
        wid = s * NC + c
        sl = pl.ds(s * NODES_PER_SUB, NODES_PER_SUB)
        pltpu.sync_copy(zeros_hbm.at[sl], acc.at[sl])
        plsc.subcore_barrier()

        def body(j, carry):
            r = wid + j * NW

            @pl.when(r < R_GROUPS)
            def _():
                pltpu.sync_copy(idx_hbm.at[r], idx_v)
                pltpu.sync_copy(rows_hbm.at[pl.ds(r * CHUNK, CHUNK)], rows_v)
                pltpu.sync_copy(rows_v, acc.at[idx_v], add=True)

            return carry

        lax.fori_loop(0, ROWS_PER_W, body, 0)
        plsc.subcore_barrier()
        pltpu.sync_copy(acc.at[sl], out_hbm.at[c, sl])

    return _scatter_kernel


def _final_body(p_ref, bnw_ref, bnb_ref, sv3_ref, r824_ref, out_ref):
    p = p_ref[...]
    sums = p[0] + p[1]
    cnt = jnp.maximum(sums[:, 32:33], 1.0)
    o = sums[:, :IN_DIM] / cnt
    s = o[:, :MUL]
    v = o[:, MUL:]
    s_mean = jnp.mean(s, axis=0, keepdims=True)
    s_c = s - s_mean
    s_var = jnp.mean(s_c * s_c, axis=0, keepdims=True)
    bnw = bnw_ref[...]
    s_out = s_c * (lax.rsqrt(s_var + EPS) * bnw[:, :MUL]) + bnb_ref[...]
    vsq = jnp.mean(v * v, axis=0, keepdims=True)
    v_norm = jnp.dot(vsq, sv3_ref[...], preferred_element_type=jnp.float32)
    vfac = lax.rsqrt(v_norm + EPS) * bnw[:, MUL:]
    vfac24 = jnp.dot(vfac, r824_ref[...], preferred_element_type=jnp.float32)
    out_ref[...] = jnp.concatenate([s_out, v * vfac24], axis=1)


_final_call = pl.pallas_call(
    _final_body,
    in_specs=[
        pl.BlockSpec((NC, N_NODES, OUT_W), lambda: (0, 0, 0)),
        pl.BlockSpec((1, 2 * MUL), lambda: (0, 0)),
        pl.BlockSpec((1, MUL), lambda: (0, 0)),
        pl.BlockSpec((24, 8), lambda: (0, 0)),
        pl.BlockSpec((8, 24), lambda: (0, 0)),
    ],
    out_specs=pl.BlockSpec((N_NODES, IN_DIM), lambda: (0, 0)),
    out_shape=jax.ShapeDtypeStruct((N_NODES, IN_DIM), jnp.float32),
)


def kernel(src_features, edge_sh, edge_scalars, edge_index, W1, b1, W2, b2,
           bn_weight, bn_bias):
    src2d = edge_index[0].reshape(R_GROUPS, CHUNK)
    dst2d = edge_index[1].reshape(R_GROUPS, CHUNK)
    win = (
        jnp.zeros((36, 56), jnp.float32)
        .at[:SCAL_DIM, :HID].set(W1)
        .at[SCAL_DIM:, HID:].set(_PRE_NP)
        .astype(jnp.bfloat16)
    )
    b_in = jnp.concatenate([b1, jnp.zeros((24,), jnp.float32)]).reshape(1, 56)
    w2f = (W2 @ _F_NP).astype(jnp.bfloat16)
    b2f = (b2 @ _F_NP).reshape(1, 512).astype(jnp.bfloat16)
    x_g = _make_gather_kernel()(src_features, src2d)
    out_tp = _edge_call(edge_scalars, x_g, edge_sh, win, b_in, w2f, b2f,
                        _EBIG_NP.astype(jnp.bfloat16),
                        _RBIG_NP.astype(jnp.bfloat16))
    partial = _make_scatter_kernel()(out_tp, dst2d,
                                     jnp.zeros((N_NODES, OUT_W), jnp.float32))
    return _final_call(partial, bn_weight.reshape(1, 2 * MUL),
                       bn_bias.reshape(1, MUL), _SV3_NP, _R824_NP)


# speedup vs baseline: 11.6513x; 1.6711x over previous
import functools

import jax
import jax.numpy as jnp
import numpy as np
from jax import lax
from jax.experimental import pallas as pl
from jax.experimental.pallas import tpu as pltpu
from jax.experimental.pallas import tpu_sc as plsc

N_NODES = 10000
N_EDGES = 320000
MUL = 8
IN_DIM = 32
SH_DIM = 4
SCAL_DIM = 32
HID = 32
EPS = 1e-5
ALPHA = 0.25
INV_SQRT3 = 1.0 / np.sqrt(3.0)

CHUNK = 128
R_GROUPS = N_EDGES // CHUNK
NC, NS = 2, 16
NW = NC * NS
ROWS_PER_W = -(-R_GROUPS // NW)

BLK = 2560
N_BLKS = N_EDGES // BLK

N_PAD = 10240
NODES_PER_SUB = N_PAD // NS


def _build_constants():
    ebig = np.zeros((64, 512), np.float32)
    f = np.zeros((256, 512), np.float32)
    rbig = np.zeros((512, 56), np.float32)
    for u in range(MUL):
        for w in range(MUL):
            ebig[u, 8 * u + w] = 1.0
            ebig[8 + u, 64 + 8 * u + w] = 1.0
            f[8 * u + w, 8 * u + w] = 1.0
            f[64 + 8 * u + w, 64 + 8 * u + w] = 1.0
            rbig[8 * u + w, w] = 1.0
            for i in range(3):
                ebig[16 + 3 * u + i, 128 + 24 * u + 3 * w + i] = 1.0
                ebig[40 + 3 * u + i, 320 + 24 * u + 3 * w + i] = 1.0
                f[128 + 8 * u + w, 128 + 24 * u + 3 * w + i] = 1.0
                f[192 + 8 * u + w, 320 + 24 * u + 3 * w + i] = 1.0
                rbig[320 + 24 * u + 3 * w + i, w] = 1.0
                rbig[64 + 8 * u + w, 8 + 3 * w + i] = 1.0
                rbig[128 + 24 * u + 3 * w + i, 32 + 3 * w + i] = 1.0
    pre = np.zeros((SH_DIM, 24), np.float32)
    for k in range(MUL):
        for i in range(3):
            pre[1 + i, 3 * k + i] = 1.0
    sv3 = np.zeros((24, 8), np.float32)
    r824 = np.zeros((8, 24), np.float32)
    for u in range(MUL):
        for i in range(3):
            sv3[3 * u + i, u] = 1.0 / 3.0
            r824[u, 3 * u + i] = 1.0
    return ebig, f, rbig, pre, sv3, r824


_EBIG_NP, _F_NP, _RBIG_NP, _PRE_NP, _SV3_NP, _R824_NP = _build_constants()


@functools.cache
def _make_gather_kernel():
    mesh = plsc.VectorSubcoreMesh(
        core_axis_name="c", subcore_axis_name="s", num_cores=NC,
        num_subcores=NS)

    @functools.partial(
        pl.kernel,
        out_type=jax.ShapeDtypeStruct((N_EDGES * IN_DIM // 128, 128),
                                      jnp.float32),
        mesh=mesh,
        compiler_params=pltpu.CompilerParams(use_tc_tiling_on_sc=False),
        scratch_types=[
            pltpu.VMEM((CHUNK,), jnp.int32),
            pltpu.VMEM((CHUNK, IN_DIM), jnp.float32),
            pltpu.SemaphoreType.DMA,
        ],
    )
    def _gather_kernel(src_hbm, idx_hbm, out_hbm, idx_v, rows_v, sem):
        wid = lax.axis_index("s") * NC + lax.axis_index("c")

        def body(j, carry):
            r = wid + j * NW

            @pl.when(r < R_GROUPS)
            def _():
                pltpu.sync_copy(idx_hbm.at[r], idx_v)
                pltpu.async_copy(src_hbm.at[idx_v], rows_v, sem).wait()
                row = (r // 20) * 640 + (r % 5) * CHUNK
                col = ((r % 20) // 5) * IN_DIM
                pltpu.sync_copy(
                    rows_v,
                    out_hbm.at[pl.ds(row, CHUNK), pl.ds(col, IN_DIM)])

            return carry

        lax.fori_loop(0, ROWS_PER_W, body, 0)

    return _gather_kernel


def _edge_body(es_ref, xg_ref, sh_ref, win_ref, bin_ref, w2f_ref, b2f_ref,
               ebig_ref, rbig_ref, out_ref):
    es = es_ref[...]
    sh = sh_ref[...]
    xg = xg_ref[...]
    xt = jnp.concatenate(
        [xg[:, 32 * k:32 * (k + 1)].T for k in range(4)], axis=1)
    cat = jnp.concatenate([es, sh], axis=0)
    t0 = jnp.dot(win_ref[...], cat.astype(jnp.bfloat16),
                 preferred_element_type=jnp.float32) + bin_ref[...]
    pre = t0[:HID]
    shv_t = t0[HID:HID + 24]
    h = pre * 0.5 * (1.0 + lax.erf(pre * np.float32(1.0 / np.sqrt(2.0))))
    wexp = jnp.dot(w2f_ref[...], h.astype(jnp.bfloat16),
                   preferred_element_type=jnp.float32).astype(jnp.bfloat16)
    wexp = wexp + b2f_ref[...]
    xs = xt[:MUL]
    xv = xt[MUL:]
    shs = sh[0:1]
    ain = jnp.concatenate(
        [xs * shs, xs, xv * shs, np.float32(INV_SQRT3) * xv * shv_t], axis=0)
    aexp = jnp.dot(ebig_ref[...], ain.astype(jnp.bfloat16),
                   preferred_element_type=jnp.float32).astype(jnp.bfloat16)
    p = aexp * wexp
    o = jnp.dot(rbig_ref[...], p, preferred_element_type=jnp.float32)
    out_s = np.float32(ALPHA) * o[0:8]
    out_v = np.float32(ALPHA) * (o[8:32] * shv_t + o[32:56])
    out = jnp.concatenate([out_s, out_v], axis=0)
    q = BLK // 4
    out_ref[...] = jnp.concatenate(
        [out[:, q * k:q * (k + 1)].T for k in range(4)], axis=1)


_edge_call = pl.pallas_call(
    _edge_body,
    grid=(N_BLKS,),
    in_specs=[
        pl.BlockSpec((SCAL_DIM, BLK), lambda i: (0, i)),
        pl.BlockSpec((BLK * IN_DIM // 128, 128), lambda i: (i, 0)),
        pl.BlockSpec((SH_DIM, BLK), lambda i: (0, i)),
        pl.BlockSpec((56, 36), lambda i: (0, 0)),
        pl.BlockSpec((56, 1), lambda i: (0, 0)),
        pl.BlockSpec((512, HID), lambda i: (0, 0)),
        pl.BlockSpec((512, 1), lambda i: (0, 0)),
        pl.BlockSpec((512, 64), lambda i: (0, 0)),
        pl.BlockSpec((56, 512), lambda i: (0, 0)),
    ],
    out_specs=pl.BlockSpec((BLK * IN_DIM // 128, 128), lambda i: (i, 0)),
    out_shape=jax.ShapeDtypeStruct((N_EDGES * IN_DIM // 128, 128),
                                   jnp.float32),
)


@functools.cache
def _make_scatter_kernel():
    mesh = plsc.VectorSubcoreMesh(
        core_axis_name="c", subcore_axis_name="s", num_cores=NC,
        num_subcores=NS)

    @functools.partial(
        pl.kernel,
        out_type=(
            jax.ShapeDtypeStruct((NC, N_PAD, IN_DIM), jnp.float32),
            jax.ShapeDtypeStruct((NC, N_PAD), jnp.float32),
        ),
        mesh=mesh,
        compiler_params=pltpu.CompilerParams(use_tc_tiling_on_sc=False),
        scratch_types=[
            pltpu.VMEM((CHUNK,), jnp.int32),
            pltpu.VMEM((CHUNK, IN_DIM), jnp.float32),
            pltpu.VMEM((CHUNK,), jnp.float32),
            pltpu.VMEM_SHARED((N_PAD, IN_DIM), jnp.float32),
            pltpu.VMEM_SHARED((N_PAD,), jnp.float32),
        ],
    )
    def _scatter_kernel(rows_hbm, idx_hbm, zeros_hbm, zeros1_hbm, out_hbm,
                        outc_hbm, idx_v, rows_v, ones_v, acc, acc_c):
        c = lax.axis_index("c")
        s = lax.axis_index("s")
        wid = s * NC + c
        sl = pl.ds(s * NODES_PER_SUB, NODES_PER_SUB)
        pltpu.sync_copy(zeros_hbm.at[sl], acc.at[sl])
        pltpu.sync_copy(zeros1_hbm.at[sl], acc_c.at[sl])

        def initones(k, carry):
            ones_v[pl.ds(k * 16, 16)] = jnp.ones((16,), jnp.float32)
            return carry

        lax.fori_loop(0, CHUNK // 16, initones, 0)
        plsc.subcore_barrier()

        def body(j, carry):
            r = wid + j * NW

            @pl.when(r < R_GROUPS)
            def _():
                pltpu.sync_copy(idx_hbm.at[r], idx_v)
                row = (r // 20) * 640 + (r % 5) * CHUNK
                col = ((r % 20) // 5) * IN_DIM
                pltpu.sync_copy(
                    rows_hbm.at[pl.ds(row, CHUNK), pl.ds(col, IN_DIM)],
                    rows_v)
                pltpu.sync_copy(rows_v, acc.at[idx_v], add=True)
                pltpu.sync_copy(ones_v, acc_c.at[idx_v], add=True)

            return carry

        lax.fori_loop(0, ROWS_PER_W, body, 0)
        plsc.subcore_barrier()
        pltpu.sync_copy(acc.at[sl], out_hbm.at[c, sl])
        pltpu.sync_copy(acc_c.at[sl], outc_hbm.at[c, sl])

    return _scatter_kernel


def _final_body(p_ref, c_ref, bnw_ref, bnb_ref, sv3_ref, r824_ref, out_ref):
    p = p_ref[...]
    sums = (p[0] + p[1])[:N_NODES]
    cnts = c_ref[...][:, :N_NODES]
    cnt = jnp.maximum(cnts[0:1] + cnts[1:2], 1.0).reshape(N_NODES, 1)
    o = sums / cnt
    s = o[:, :MUL]
    v = o[:, MUL:]
    s_mean = jnp.mean(s, axis=0, keepdims=True)
    s_c = s - s_mean
    s_var = jnp.mean(s_c * s_c, axis=0, keepdims=True)
    bnw = bnw_ref[...]
    s_out = s_c * (lax.rsqrt(s_var + EPS) * bnw[:, :MUL]) + bnb_ref[...]
    vsq = jnp.mean(v * v, axis=0, keepdims=True)
    v_norm = jnp.dot(vsq, sv3_ref[...], preferred_element_type=jnp.float32)
    vfac = lax.rsqrt(v_norm + EPS) * bnw[:, MUL:]
    vfac24 = jnp.dot(vfac, r824_ref[...], preferred_element_type=jnp.float32)
    out_ref[...] = jnp.concatenate([s_out, v * vfac24], axis=1)


_final_call = pl.pallas_call(
    _final_body,
    in_specs=[
        pl.BlockSpec((NC, N_PAD, IN_DIM), lambda: (0, 0, 0)),
        pl.BlockSpec((NC, N_PAD), lambda: (0, 0)),
        pl.BlockSpec((1, 2 * MUL), lambda: (0, 0)),
        pl.BlockSpec((1, MUL), lambda: (0, 0)),
        pl.BlockSpec((24, 8), lambda: (0, 0)),
        pl.BlockSpec((8, 24), lambda: (0, 0)),
    ],
    out_specs=pl.BlockSpec((N_NODES, IN_DIM), lambda: (0, 0)),
    out_shape=jax.ShapeDtypeStruct((N_NODES, IN_DIM), jnp.float32),
)


def kernel(src_features, edge_sh, edge_scalars, edge_index, W1, b1, W2, b2,
           bn_weight, bn_bias):
    src2d = edge_index[0].reshape(R_GROUPS, CHUNK)
    dst2d = edge_index[1].reshape(R_GROUPS, CHUNK)
    win = (
        jnp.zeros((36, 56), jnp.float32)
        .at[:SCAL_DIM, :HID].set(W1)
        .at[SCAL_DIM:, HID:].set(_PRE_NP)
        .T.astype(jnp.bfloat16)
    )
    b_in = jnp.concatenate([b1, jnp.zeros((24,), jnp.float32)]).reshape(56, 1)
    w2f = (W2 @ _F_NP).T.astype(jnp.bfloat16)
    b2f = (b2 @ _F_NP).reshape(512, 1).astype(jnp.bfloat16)
    xg128 = _make_gather_kernel()(src_features, src2d)
    out_tp = _edge_call(edge_scalars.T, xg128, edge_sh.T, win, b_in, w2f, b2f,
                        _EBIG_NP.T.astype(jnp.bfloat16),
                        _RBIG_NP.T.astype(jnp.bfloat16))
    partial, cnts = _make_scatter_kernel()(
        out_tp, dst2d, jnp.zeros((N_PAD, IN_DIM), jnp.float32),
        jnp.zeros((N_PAD,), jnp.float32))
    return _final_call(partial, cnts, bn_weight.reshape(1, 2 * MUL),
                       bn_bias.reshape(1, MUL), _SV3_NP, _R824_NP)


# speedup vs baseline: 14.8924x; 1.2782x over previous
import functools

import jax
import jax.numpy as jnp
import numpy as np
from jax import lax
from jax.experimental import pallas as pl
from jax.experimental.pallas import tpu as pltpu
from jax.experimental.pallas import tpu_sc as plsc

N_NODES = 10000
N_EDGES = 320000
MUL = 8
IN_DIM = 32
SH_DIM = 4
SCAL_DIM = 32
HID = 32
EPS = 1e-5
ALPHA = 0.25
INV_SQRT3 = 1.0 / np.sqrt(3.0)

CHUNK = 640
R_GROUPS = N_EDGES // CHUNK
NC, NS = 2, 16
NW = NC * NS
ROWS_PER_W = -(-R_GROUPS // NW)

BLK = 2560
N_BLKS = N_EDGES // BLK

N_PAD = 10240
NODES_PER_SUB = N_PAD // NS


def _build_constants():
    ebig = np.zeros((64, 512), np.float32)
    f = np.zeros((256, 512), np.float32)
    rbig = np.zeros((512, 56), np.float32)
    for u in range(MUL):
        for w in range(MUL):
            ebig[u, 8 * u + w] = 1.0
            ebig[8 + u, 64 + 8 * u + w] = 1.0
            f[8 * u + w, 8 * u + w] = 1.0
            f[64 + 8 * u + w, 64 + 8 * u + w] = 1.0
            rbig[8 * u + w, w] = 1.0
            for i in range(3):
                ebig[16 + 3 * u + i, 128 + 24 * u + 3 * w + i] = 1.0
                ebig[40 + 3 * u + i, 320 + 24 * u + 3 * w + i] = 1.0
                f[128 + 8 * u + w, 128 + 24 * u + 3 * w + i] = 1.0
                f[192 + 8 * u + w, 320 + 24 * u + 3 * w + i] = 1.0
                rbig[320 + 24 * u + 3 * w + i, w] = 1.0
                rbig[64 + 8 * u + w, 8 + 3 * w + i] = 1.0
                rbig[128 + 24 * u + 3 * w + i, 32 + 3 * w + i] = 1.0
    pre = np.zeros((SH_DIM, 24), np.float32)
    for k in range(MUL):
        for i in range(3):
            pre[1 + i, 3 * k + i] = 1.0
    sv3 = np.zeros((24, 8), np.float32)
    r824 = np.zeros((8, 24), np.float32)
    for u in range(MUL):
        for i in range(3):
            sv3[3 * u + i, u] = 1.0 / 3.0
            r824[u, 3 * u + i] = 1.0
    return ebig, f, rbig, pre, sv3, r824


_EBIG_NP, _F_NP, _RBIG_NP, _PRE_NP, _SV3_NP, _R824_NP = _build_constants()


@functools.cache
def _make_gather_kernel():
    mesh = plsc.VectorSubcoreMesh(
        core_axis_name="c", subcore_axis_name="s", num_cores=NC,
        num_subcores=NS)

    @functools.partial(
        pl.kernel,
        out_type=jax.ShapeDtypeStruct((N_EDGES * IN_DIM // 128, 128),
                                      jnp.float32),
        mesh=mesh,
        compiler_params=pltpu.CompilerParams(use_tc_tiling_on_sc=False),
        scratch_types=[
            pltpu.VMEM((CHUNK,), jnp.int32),
            pltpu.VMEM((CHUNK, IN_DIM), jnp.float32),
            pltpu.SemaphoreType.DMA,
        ],
    )
    def _gather_kernel(src_hbm, idx_hbm, out_hbm, idx_v, rows_v, sem):
        wid = lax.axis_index("s") * NC + lax.axis_index("c")

        def body(j, carry):
            r = wid + j * NW

            @pl.when(r < R_GROUPS)
            def _():
                pltpu.sync_copy(idx_hbm.at[r], idx_v)
                pltpu.async_copy(src_hbm.at[idx_v], rows_v, sem).wait()
                row = (r // 4) * 640
                col = (r % 4) * IN_DIM
                pltpu.sync_copy(
                    rows_v,
                    out_hbm.at[pl.ds(row, 640), pl.ds(col, IN_DIM)])

            return carry

        lax.fori_loop(0, ROWS_PER_W, body, 0)

    return _gather_kernel


def _edge_body(es_ref, xg_ref, sh_ref, win_ref, bin_ref, w2f_ref, b2f_ref,
               ebig_ref, rbig_ref, out_ref):
    es = es_ref[...]
    sh = sh_ref[...]
    xg = xg_ref[...]
    xt = jnp.concatenate(
        [xg[:, 32 * k:32 * (k + 1)].T for k in range(4)], axis=1)
    cat = jnp.concatenate([es, sh], axis=0)
    t0 = jnp.dot(win_ref[...], cat.astype(jnp.bfloat16),
                 preferred_element_type=jnp.float32) + bin_ref[...]
    pre = t0[:HID]
    shv_t = t0[HID:HID + 24]
    h = pre * 0.5 * (1.0 + lax.erf(pre * np.float32(1.0 / np.sqrt(2.0))))
    wexp = jnp.dot(w2f_ref[...], h.astype(jnp.bfloat16),
                   preferred_element_type=jnp.float32).astype(jnp.bfloat16)
    wexp = wexp + b2f_ref[...]
    xs = xt[:MUL]
    xv = xt[MUL:]
    shs = sh[0:1]
    ain = jnp.concatenate(
        [xs * shs, xs, xv * shs, np.float32(INV_SQRT3) * xv * shv_t], axis=0)
    aexp = jnp.dot(ebig_ref[...], ain.astype(jnp.bfloat16),
                   preferred_element_type=jnp.float32).astype(jnp.bfloat16)
    p = aexp * wexp
    o = jnp.dot(rbig_ref[...], p, preferred_element_type=jnp.float32)
    out_s = np.float32(ALPHA) * o[0:8]
    out_v = np.float32(ALPHA) * (o[8:32] * shv_t + o[32:56])
    out = jnp.concatenate([out_s, out_v], axis=0)
    q = BLK // 4
    out_ref[...] = jnp.concatenate(
        [out[:, q * k:q * (k + 1)].T for k in range(4)], axis=1)


_edge_call = pl.pallas_call(
    _edge_body,
    grid=(N_BLKS,),
    in_specs=[
        pl.BlockSpec((SCAL_DIM, BLK), lambda i: (0, i)),
        pl.BlockSpec((BLK * IN_DIM // 128, 128), lambda i: (i, 0)),
        pl.BlockSpec((SH_DIM, BLK), lambda i: (0, i)),
        pl.BlockSpec((56, 36), lambda i: (0, 0)),
        pl.BlockSpec((56, 1), lambda i: (0, 0)),
        pl.BlockSpec((512, HID), lambda i: (0, 0)),
        pl.BlockSpec((512, 1), lambda i: (0, 0)),
        pl.BlockSpec((512, 64), lambda i: (0, 0)),
        pl.BlockSpec((56, 512), lambda i: (0, 0)),
    ],
    out_specs=pl.BlockSpec((BLK * IN_DIM // 128, 128), lambda i: (i, 0)),
    out_shape=jax.ShapeDtypeStruct((N_EDGES * IN_DIM // 128, 128),
                                   jnp.float32),
)


@functools.cache
def _make_scatter_kernel():
    mesh = plsc.VectorSubcoreMesh(
        core_axis_name="c", subcore_axis_name="s", num_cores=NC,
        num_subcores=NS)

    @functools.partial(
        pl.kernel,
        out_type=(
            jax.ShapeDtypeStruct((NC, N_PAD, IN_DIM), jnp.float32),
            jax.ShapeDtypeStruct((NC, N_PAD), jnp.float32),
        ),
        mesh=mesh,
        compiler_params=pltpu.CompilerParams(use_tc_tiling_on_sc=False),
        scratch_types=[
            pltpu.VMEM((CHUNK,), jnp.int32),
            pltpu.VMEM((CHUNK, IN_DIM), jnp.float32),
            pltpu.VMEM((CHUNK,), jnp.float32),
            pltpu.VMEM_SHARED((N_PAD, IN_DIM), jnp.float32),
            pltpu.VMEM_SHARED((N_PAD,), jnp.float32),
        ],
    )
    def _scatter_kernel(rows_hbm, idx_hbm, zeros_hbm, zeros1_hbm, out_hbm,
                        outc_hbm, idx_v, rows_v, ones_v, acc, acc_c):
        c = lax.axis_index("c")
        s = lax.axis_index("s")
        wid = s * NC + c
        sl = pl.ds(s * NODES_PER_SUB, NODES_PER_SUB)
        pltpu.sync_copy(zeros_hbm.at[sl], acc.at[sl])
        pltpu.sync_copy(zeros1_hbm.at[sl], acc_c.at[sl])

        def initones(k, carry):
            ones_v[pl.ds(k * 16, 16)] = jnp.ones((16,), jnp.float32)
            return carry

        lax.fori_loop(0, CHUNK // 16, initones, 0)
        plsc.subcore_barrier()

        def body(j, carry):
            r = wid + j * NW

            @pl.when(r < R_GROUPS)
            def _():
                pltpu.sync_copy(idx_hbm.at[r], idx_v)
                row = (r // 4) * 640
                col = (r % 4) * IN_DIM
                pltpu.sync_copy(
                    rows_hbm.at[pl.ds(row, 640), pl.ds(col, IN_DIM)],
                    rows_v)
                pltpu.sync_copy(rows_v, acc.at[idx_v], add=True)
                pltpu.sync_copy(ones_v, acc_c.at[idx_v], add=True)

            return carry

        lax.fori_loop(0, ROWS_PER_W, body, 0)
        plsc.subcore_barrier()
        pltpu.sync_copy(acc.at[sl], out_hbm.at[c, sl])
        pltpu.sync_copy(acc_c.at[sl], outc_hbm.at[c, sl])

    return _scatter_kernel


def _final_body(p_ref, c_ref, bnw_ref, bnb_ref, sv3_ref, r824_ref, out_ref):
    p = p_ref[...]
    sums = (p[0] + p[1])[:N_NODES]
    cnts = c_ref[...][:, :N_NODES]
    cnt = jnp.maximum(cnts[0:1] + cnts[1:2], 1.0).reshape(N_NODES, 1)
    o = sums / cnt
    s = o[:, :MUL]
    v = o[:, MUL:]
    s_mean = jnp.mean(s, axis=0, keepdims=True)
    s_c = s - s_mean
    s_var = jnp.mean(s_c * s_c, axis=0, keepdims=True)
    bnw = bnw_ref[...]
    s_out = s_c * (lax.rsqrt(s_var + EPS) * bnw[:, :MUL]) + bnb_ref[...]
    vsq = jnp.mean(v * v, axis=0, keepdims=True)
    v_norm = jnp.dot(vsq, sv3_ref[...], preferred_element_type=jnp.float32)
    vfac = lax.rsqrt(v_norm + EPS) * bnw[:, MUL:]
    vfac24 = jnp.dot(vfac, r824_ref[...], preferred_element_type=jnp.float32)
    out_ref[...] = jnp.concatenate([s_out, v * vfac24], axis=1)


_final_call = pl.pallas_call(
    _final_body,
    in_specs=[
        pl.BlockSpec((NC, N_PAD, IN_DIM), lambda: (0, 0, 0)),
        pl.BlockSpec((NC, N_PAD), lambda: (0, 0)),
        pl.BlockSpec((1, 2 * MUL), lambda: (0, 0)),
        pl.BlockSpec((1, MUL), lambda: (0, 0)),
        pl.BlockSpec((24, 8), lambda: (0, 0)),
        pl.BlockSpec((8, 24), lambda: (0, 0)),
    ],
    out_specs=pl.BlockSpec((N_NODES, IN_DIM), lambda: (0, 0)),
    out_shape=jax.ShapeDtypeStruct((N_NODES, IN_DIM), jnp.float32),
)


def kernel(src_features, edge_sh, edge_scalars, edge_index, W1, b1, W2, b2,
           bn_weight, bn_bias):
    src2d = edge_index[0].reshape(R_GROUPS, CHUNK)
    dst2d = edge_index[1].reshape(R_GROUPS, CHUNK)
    win = (
        jnp.zeros((36, 56), jnp.float32)
        .at[:SCAL_DIM, :HID].set(W1)
        .at[SCAL_DIM:, HID:].set(_PRE_NP)
        .T.astype(jnp.bfloat16)
    )
    b_in = jnp.concatenate([b1, jnp.zeros((24,), jnp.float32)]).reshape(56, 1)
    w2f = (W2 @ _F_NP).T.astype(jnp.bfloat16)
    b2f = (b2 @ _F_NP).reshape(512, 1).astype(jnp.bfloat16)
    xg128 = _make_gather_kernel()(src_features, src2d)
    out_tp = _edge_call(edge_scalars.T, xg128, edge_sh.T, win, b_in, w2f, b2f,
                        _EBIG_NP.T.astype(jnp.bfloat16),
                        _RBIG_NP.T.astype(jnp.bfloat16))
    partial, cnts = _make_scatter_kernel()(
        out_tp, dst2d, jnp.zeros((N_PAD, IN_DIM), jnp.float32),
        jnp.zeros((N_PAD,), jnp.float32))
    return _final_call(partial, cnts, bn_weight.reshape(1, 2 * MUL),
                       bn_bias.reshape(1, MUL), _SV3_NP, _R824_NP)


# speedup vs baseline: 15.9493x; 1.0710x over previous
import functools

import jax
import jax.numpy as jnp
import numpy as np
from jax import lax
from jax.experimental import pallas as pl
from jax.experimental.pallas import tpu as pltpu
from jax.experimental.pallas import tpu_sc as plsc

N_NODES = 10000
N_EDGES = 320000
MUL = 8
IN_DIM = 32
SH_DIM = 4
SCAL_DIM = 32
HID = 32
EPS = 1e-5
ALPHA = 0.25
INV_SQRT3 = 1.0 / np.sqrt(3.0)

CHUNK = 640
R_GROUPS = N_EDGES // CHUNK
NC, NS = 2, 16
NW = NC * NS
ROWS_PER_W = -(-R_GROUPS // NW)

BLK = 2560
N_BLKS = N_EDGES // BLK

N_PAD = 10240
NODES_PER_SUB = N_PAD // NS


def _build_constants():
    ebig = np.zeros((48, 384), np.float32)
    f = np.zeros((256, 384), np.float32)
    rbig = np.zeros((384, 56), np.float32)
    for u in range(MUL):
        for w in range(MUL):
            ebig[u, 8 * u + w] = 1.0
            ebig[8 + u, 64 + 8 * u + w] = 1.0
            ebig[40 + u, 320 + 8 * u + w] = 1.0
            f[8 * u + w, 8 * u + w] = 1.0
            f[64 + 8 * u + w, 64 + 8 * u + w] = 1.0
            f[192 + 8 * u + w, 320 + 8 * u + w] = 1.0
            rbig[8 * u + w, w] = 1.0
            rbig[320 + 8 * u + w, w] = 1.0
            for i in range(3):
                ebig[16 + 3 * u + i, 128 + 24 * u + 3 * w + i] = 1.0
                f[128 + 8 * u + w, 128 + 24 * u + 3 * w + i] = 1.0
                rbig[64 + 8 * u + w, 8 + 3 * w + i] = 1.0
                rbig[128 + 24 * u + 3 * w + i, 32 + 3 * w + i] = 1.0
    pre = np.zeros((SH_DIM, 24), np.float32)
    for k in range(MUL):
        for i in range(3):
            pre[1 + i, 3 * k + i] = 1.0
    sv3 = np.zeros((24, 8), np.float32)
    r824 = np.zeros((8, 24), np.float32)
    sv1 = np.zeros((8, 24), np.float32)
    for u in range(MUL):
        for i in range(3):
            sv3[3 * u + i, u] = 1.0 / 3.0
            r824[u, 3 * u + i] = 1.0
            sv1[u, 3 * u + i] = 1.0
    return ebig, f, rbig, pre, sv3, r824, sv1


(_EBIG_NP, _F_NP, _RBIG_NP, _PRE_NP, _SV3_NP, _R824_NP,
 _SV1_NP) = _build_constants()


@functools.cache
def _make_gather_kernel():
    mesh = plsc.VectorSubcoreMesh(
        core_axis_name="c", subcore_axis_name="s", num_cores=NC,
        num_subcores=NS)

    @functools.partial(
        pl.kernel,
        out_type=jax.ShapeDtypeStruct((N_EDGES * IN_DIM // 128, 128),
                                      jnp.float32),
        mesh=mesh,
        compiler_params=pltpu.CompilerParams(use_tc_tiling_on_sc=False),
        scratch_types=[
            pltpu.VMEM((CHUNK,), jnp.int32),
            pltpu.VMEM((CHUNK, IN_DIM), jnp.float32),
            pltpu.SemaphoreType.DMA,
        ],
    )
    def _gather_kernel(src_hbm, idx_hbm, out_hbm, idx_v, rows_v, sem):
        wid = lax.axis_index("s") * NC + lax.axis_index("c")

        def body(j, carry):
            r = wid + j * NW

            @pl.when(r < R_GROUPS)
            def _():
                pltpu.sync_copy(idx_hbm.at[r], idx_v)
                pltpu.async_copy(src_hbm.at[idx_v], rows_v, sem).wait()
                row = (r // 4) * 640
                col = (r % 4) * IN_DIM
                pltpu.sync_copy(
                    rows_v,
                    out_hbm.at[pl.ds(row, 640), pl.ds(col, IN_DIM)])

            return carry

        lax.fori_loop(0, ROWS_PER_W, body, 0)

    return _gather_kernel


def _edge_body(es_ref, xg_ref, sh_ref, win_ref, bin_ref, w2f_ref, b2f_ref,
               ebig_ref, rbig_ref, sv1_ref, out_ref):
    es = es_ref[...]
    sh = sh_ref[...]
    xg = xg_ref[...]
    xt = jnp.concatenate(
        [xg[:, 32 * k:32 * (k + 1)].T for k in range(4)], axis=1)
    cat = jnp.concatenate([es, sh], axis=0)
    t0 = jnp.dot(win_ref[...], cat.astype(jnp.bfloat16),
                 preferred_element_type=jnp.float32) + bin_ref[...]
    pre = t0[:HID]
    shv_t = t0[HID:HID + 24]
    h = pre * 0.5 * (1.0 + lax.erf(pre * np.float32(1.0 / np.sqrt(2.0))))
    wexp = jnp.dot(w2f_ref[...], h.astype(jnp.bfloat16),
                   preferred_element_type=jnp.float32).astype(jnp.bfloat16)
    wexp = wexp + b2f_ref[...]
    xs = xt[:MUL]
    xv = xt[MUL:]
    shs = sh[0:1]
    dotvv = jnp.dot(sv1_ref[...], (xv * shv_t).astype(jnp.bfloat16),
                    preferred_element_type=jnp.float32)
    ain = jnp.concatenate(
        [xs * shs, xs, xv * shs, np.float32(INV_SQRT3) * dotvv], axis=0)
    aexp = jnp.dot(ebig_ref[...], ain.astype(jnp.bfloat16),
                   preferred_element_type=jnp.float32).astype(jnp.bfloat16)
    p = aexp * wexp
    o = jnp.dot(rbig_ref[...], p, preferred_element_type=jnp.float32)
    out_s = np.float32(ALPHA) * o[0:8]
    out_v = np.float32(ALPHA) * (o[8:32] * shv_t + o[32:56])
    out = jnp.concatenate([out_s, out_v], axis=0)
    q = BLK // 4
    out_ref[...] = jnp.concatenate(
        [out[:, q * k:q * (k + 1)].T for k in range(4)], axis=1)


_edge_call = pl.pallas_call(
    _edge_body,
    grid=(N_BLKS,),
    in_specs=[
        pl.BlockSpec((SCAL_DIM, BLK), lambda i: (0, i)),
        pl.BlockSpec((BLK * IN_DIM // 128, 128), lambda i: (i, 0)),
        pl.BlockSpec((SH_DIM, BLK), lambda i: (0, i)),
        pl.BlockSpec((56, 36), lambda i: (0, 0)),
        pl.BlockSpec((56, 1), lambda i: (0, 0)),
        pl.BlockSpec((384, HID), lambda i: (0, 0)),
        pl.BlockSpec((384, 1), lambda i: (0, 0)),
        pl.BlockSpec((384, 48), lambda i: (0, 0)),
        pl.BlockSpec((56, 384), lambda i: (0, 0)),
        pl.BlockSpec((8, 24), lambda i: (0, 0)),
    ],
    out_specs=pl.BlockSpec((BLK * IN_DIM // 128, 128), lambda i: (i, 0)),
    out_shape=jax.ShapeDtypeStruct((N_EDGES * IN_DIM // 128, 128),
                                   jnp.float32),
)


@functools.cache
def _make_scatter_kernel():
    mesh = plsc.VectorSubcoreMesh(
        core_axis_name="c", subcore_axis_name="s", num_cores=NC,
        num_subcores=NS)

    @functools.partial(
        pl.kernel,
        out_type=(
            jax.ShapeDtypeStruct((NC, N_PAD, IN_DIM), jnp.float32),
            jax.ShapeDtypeStruct((NC, N_PAD), jnp.float32),
        ),
        mesh=mesh,
        compiler_params=pltpu.CompilerParams(use_tc_tiling_on_sc=False),
        scratch_types=[
            pltpu.VMEM((CHUNK,), jnp.int32),
            pltpu.VMEM((CHUNK, IN_DIM), jnp.float32),
            pltpu.VMEM((CHUNK,), jnp.float32),
            pltpu.VMEM_SHARED((N_PAD, IN_DIM), jnp.float32),
            pltpu.VMEM_SHARED((N_PAD,), jnp.float32),
        ],
    )
    def _scatter_kernel(rows_hbm, idx_hbm, zeros_hbm, zeros1_hbm, out_hbm,
                        outc_hbm, idx_v, rows_v, ones_v, acc, acc_c):
        c = lax.axis_index("c")
        s = lax.axis_index("s")
        wid = s * NC + c
        sl = pl.ds(s * NODES_PER_SUB, NODES_PER_SUB)
        pltpu.sync_copy(zeros_hbm.at[sl], acc.at[sl])
        pltpu.sync_copy(zeros1_hbm.at[sl], acc_c.at[sl])

        def initones(k, carry):
            ones_v[pl.ds(k * 16, 16)] = jnp.ones((16,), jnp.float32)
            return carry

        lax.fori_loop(0, CHUNK // 16, initones, 0)
        plsc.subcore_barrier()

        def body(j, carry):
            r = wid + j * NW

            @pl.when(r < R_GROUPS)
            def _():
                pltpu.sync_copy(idx_hbm.at[r], idx_v)
                row = (r // 4) * 640
                col = (r % 4) * IN_DIM
                pltpu.sync_copy(
                    rows_hbm.at[pl.ds(row, 640), pl.ds(col, IN_DIM)],
                    rows_v)
                pltpu.sync_copy(rows_v, acc.at[idx_v], add=True)
                pltpu.sync_copy(ones_v, acc_c.at[idx_v], add=True)

            return carry

        lax.fori_loop(0, ROWS_PER_W, body, 0)
        plsc.subcore_barrier()
        pltpu.sync_copy(acc.at[sl], out_hbm.at[c, sl])
        pltpu.sync_copy(acc_c.at[sl], outc_hbm.at[c, sl])

    return _scatter_kernel


def _final_body(p_ref, c_ref, bnw_ref, bnb_ref, sv3_ref, r824_ref, out_ref):
    p = p_ref[...]
    sums = (p[0] + p[1])[:N_NODES]
    cnts = c_ref[...][:, :N_NODES]
    cnt = jnp.maximum(cnts[0:1] + cnts[1:2], 1.0).reshape(N_NODES, 1)
    o = sums / cnt
    s = o[:, :MUL]
    v = o[:, MUL:]
    s_mean = jnp.mean(s, axis=0, keepdims=True)
    s_c = s - s_mean
    s_var = jnp.mean(s_c * s_c, axis=0, keepdims=True)
    bnw = bnw_ref[...]
    s_out = s_c * (lax.rsqrt(s_var + EPS) * bnw[:, :MUL]) + bnb_ref[...]
    vsq = jnp.mean(v * v, axis=0, keepdims=True)
    v_norm = jnp.dot(vsq, sv3_ref[...], preferred_element_type=jnp.float32)
    vfac = lax.rsqrt(v_norm + EPS) * bnw[:, MUL:]
    vfac24 = jnp.dot(vfac, r824_ref[...], preferred_element_type=jnp.float32)
    out_ref[...] = jnp.concatenate([s_out, v * vfac24], axis=1)


_final_call = pl.pallas_call(
    _final_body,
    in_specs=[
        pl.BlockSpec((NC, N_PAD, IN_DIM), lambda: (0, 0, 0)),
        pl.BlockSpec((NC, N_PAD), lambda: (0, 0)),
        pl.BlockSpec((1, 2 * MUL), lambda: (0, 0)),
        pl.BlockSpec((1, MUL), lambda: (0, 0)),
        pl.BlockSpec((24, 8), lambda: (0, 0)),
        pl.BlockSpec((8, 24), lambda: (0, 0)),
    ],
    out_specs=pl.BlockSpec((N_NODES, IN_DIM), lambda: (0, 0)),
    out_shape=jax.ShapeDtypeStruct((N_NODES, IN_DIM), jnp.float32),
)


def kernel(src_features, edge_sh, edge_scalars, edge_index, W1, b1, W2, b2,
           bn_weight, bn_bias):
    src2d = edge_index[0].reshape(R_GROUPS, CHUNK)
    dst2d = edge_index[1].reshape(R_GROUPS, CHUNK)
    win = (
        jnp.zeros((36, 56), jnp.float32)
        .at[:SCAL_DIM, :HID].set(W1)
        .at[SCAL_DIM:, HID:].set(_PRE_NP)
        .T.astype(jnp.bfloat16)
    )
    b_in = jnp.concatenate([b1, jnp.zeros((24,), jnp.float32)]).reshape(56, 1)
    w2f = (W2 @ _F_NP).T.astype(jnp.bfloat16)
    b2f = (b2 @ _F_NP).reshape(384, 1).astype(jnp.bfloat16)
    xg128 = _make_gather_kernel()(src_features, src2d)
    out_tp = _edge_call(edge_scalars.T, xg128, edge_sh.T, win, b_in, w2f, b2f,
                        _EBIG_NP.T.astype(jnp.bfloat16),
                        _RBIG_NP.T.astype(jnp.bfloat16),
                        _SV1_NP.astype(jnp.bfloat16))
    partial, cnts = _make_scatter_kernel()(
        out_tp, dst2d, jnp.zeros((N_PAD, IN_DIM), jnp.float32),
        jnp.zeros((N_PAD,), jnp.float32))
    return _final_call(partial, cnts, bn_weight.reshape(1, 2 * MUL),
                       bn_bias.reshape(1, MUL), _SV3_NP, _R824_NP)


# speedup vs baseline: 16.9378x; 1.0620x over previous
import functools

import jax
import jax.numpy as jnp
import numpy as np
from jax import lax
from jax.experimental import pallas as pl
from jax.experimental.pallas import tpu as pltpu
from jax.experimental.pallas import tpu_sc as plsc

N_NODES = 10000
N_EDGES = 320000
MUL = 8
IN_DIM = 32
SH_DIM = 4
SCAL_DIM = 32
HID = 32
EPS = 1e-5
ALPHA = 0.25
INV_SQRT3 = 1.0 / np.sqrt(3.0)

CHUNK = 640
R_GROUPS = N_EDGES // CHUNK
NC, NS = 2, 16
NW = NC * NS
ROWS_PER_W = -(-R_GROUPS // NW)

BLK = 2560
N_BLKS = N_EDGES // BLK

N_PAD = 10240
NODES_PER_SUB = N_PAD // NS


def _build_constants():
    ebig = np.zeros((48, 384), np.float32)
    f = np.zeros((256, 384), np.float32)
    rbig = np.zeros((384, 56), np.float32)
    for u in range(MUL):
        for w in range(MUL):
            ebig[u, 8 * u + w] = 1.0
            ebig[8 + u, 64 + 8 * u + w] = 1.0
            ebig[40 + u, 320 + 8 * u + w] = 1.0
            f[8 * u + w, 8 * u + w] = 1.0
            f[64 + 8 * u + w, 64 + 8 * u + w] = 1.0
            f[192 + 8 * u + w, 320 + 8 * u + w] = 1.0
            rbig[8 * u + w, w] = 1.0
            rbig[320 + 8 * u + w, w] = 1.0
            for i in range(3):
                ebig[16 + 3 * u + i, 128 + 24 * u + 3 * w + i] = 1.0
                f[128 + 8 * u + w, 128 + 24 * u + 3 * w + i] = 1.0
                rbig[64 + 8 * u + w, 8 + 3 * w + i] = 1.0
                rbig[128 + 24 * u + 3 * w + i, 32 + 3 * w + i] = 1.0
    pre = np.zeros((SH_DIM, 24), np.float32)
    for k in range(MUL):
        for i in range(3):
            pre[1 + i, 3 * k + i] = 1.0
    sv3 = np.zeros((24, 8), np.float32)
    r824 = np.zeros((8, 24), np.float32)
    sv1 = np.zeros((8, 24), np.float32)
    for u in range(MUL):
        for i in range(3):
            sv3[3 * u + i, u] = 1.0 / 3.0
            r824[u, 3 * u + i] = 1.0
            sv1[u, 3 * u + i] = 1.0
    return ebig, f, rbig, pre, sv3, r824, sv1


(_EBIG_NP, _F_NP, _RBIG_NP, _PRE_NP, _SV3_NP, _R824_NP,
 _SV1_NP) = _build_constants()


@functools.cache
def _make_gather_kernel():
    mesh = plsc.VectorSubcoreMesh(
        core_axis_name="c", subcore_axis_name="s", num_cores=NC,
        num_subcores=NS)

    @functools.partial(
        pl.kernel,
        out_type=jax.ShapeDtypeStruct((N_EDGES * IN_DIM // 128, 128),
                                      jnp.float32),
        mesh=mesh,
        compiler_params=pltpu.CompilerParams(use_tc_tiling_on_sc=False),
        scratch_types=[
            pltpu.VMEM((2, CHUNK), jnp.int32),
            pltpu.VMEM((2, CHUNK, IN_DIM), jnp.float32),
            pltpu.SemaphoreType.DMA,
            pltpu.SemaphoreType.DMA,
            pltpu.SemaphoreType.DMA,
        ],
    )
    def _gather_kernel(src_hbm, idx_hbm, out_hbm, idx_v, rows_v, sem_i, sem_g,
                       sem_o):
        wid = lax.axis_index("s") * NC + lax.axis_index("c")

        def dst_slice(r):
            return out_hbm.at[pl.ds((r // 4) * 640, 640),
                              pl.ds((r % 4) * IN_DIM, IN_DIM)]

        n_full = R_GROUPS // NW
        di = pltpu.async_copy(idx_hbm.at[wid], idx_v.at[0], sem_i)
        wo = {}
        for j in range(n_full):
            b = j % 2
            r = wid + j * NW
            if j + 1 < n_full:
                di_next = pltpu.async_copy(
                    idx_hbm.at[wid + (j + 1) * NW], idx_v.at[1 - b], sem_i)
            di.wait()
            if j >= 2:
                wo[j - 2].wait()
            pltpu.async_copy(src_hbm.at[idx_v.at[b]], rows_v.at[b],
                             sem_g).wait()
            wo[j] = pltpu.async_copy(rows_v.at[b], dst_slice(r), sem_o)
            if j + 1 < n_full:
                di = di_next
        wo[n_full - 2].wait()
        wo[n_full - 1].wait()
        r_tail = wid + n_full * NW

        @pl.when(r_tail < R_GROUPS)
        def _():
            pltpu.sync_copy(idx_hbm.at[r_tail], idx_v.at[0])
            pltpu.async_copy(src_hbm.at[idx_v.at[0]], rows_v.at[0],
                             sem_g).wait()
            pltpu.sync_copy(rows_v.at[0], dst_slice(r_tail))

    return _gather_kernel


def _edge_body(es_ref, xg_ref, sh_ref, win_ref, bin_ref, w2f_ref, b2f_ref,
               ebig_ref, rbig_ref, sv1_ref, out_ref):
    es = es_ref[...]
    sh = sh_ref[...]
    xg = xg_ref[...]
    xt = jnp.concatenate(
        [xg[:, 32 * k:32 * (k + 1)].T for k in range(4)], axis=1)
    cat = jnp.concatenate([es, sh], axis=0)
    t0 = jnp.dot(win_ref[...], cat.astype(jnp.bfloat16),
                 preferred_element_type=jnp.float32) + bin_ref[...]
    pre = t0[:HID]
    shv_t = t0[HID:HID + 24]
    h = pre * 0.5 * (1.0 + lax.erf(pre * np.float32(1.0 / np.sqrt(2.0))))
    wexp = jnp.dot(w2f_ref[...], h.astype(jnp.bfloat16),
                   preferred_element_type=jnp.float32).astype(jnp.bfloat16)
    wexp = wexp + b2f_ref[...]
    xs = xt[:MUL]
    xv = xt[MUL:]
    shs = sh[0:1]
    dotvv = jnp.dot(sv1_ref[...], (xv * shv_t).astype(jnp.bfloat16),
                    preferred_element_type=jnp.float32)
    ain = jnp.concatenate(
        [xs * shs, xs, xv * shs, np.float32(INV_SQRT3) * dotvv], axis=0)
    aexp = jnp.dot(ebig_ref[...], ain.astype(jnp.bfloat16),
                   preferred_element_type=jnp.float32).astype(jnp.bfloat16)
    p = aexp * wexp
    o = jnp.dot(rbig_ref[...], p, preferred_element_type=jnp.float32)
    out_s = np.float32(ALPHA) * o[0:8]
    out_v = np.float32(ALPHA) * (o[8:32] * shv_t + o[32:56])
    out = jnp.concatenate([out_s, out_v], axis=0)
    q = BLK // 4
    out_ref[...] = jnp.concatenate(
        [out[:, q * k:q * (k + 1)].T for k in range(4)], axis=1)


_edge_call = pl.pallas_call(
    _edge_body,
    grid=(N_BLKS,),
    in_specs=[
        pl.BlockSpec((SCAL_DIM, BLK), lambda i: (0, i)),
        pl.BlockSpec((BLK * IN_DIM // 128, 128), lambda i: (i, 0)),
        pl.BlockSpec((SH_DIM, BLK), lambda i: (0, i)),
        pl.BlockSpec((56, 36), lambda i: (0, 0)),
        pl.BlockSpec((56, 1), lambda i: (0, 0)),
        pl.BlockSpec((384, HID), lambda i: (0, 0)),
        pl.BlockSpec((384, 1), lambda i: (0, 0)),
        pl.BlockSpec((384, 48), lambda i: (0, 0)),
        pl.BlockSpec((56, 384), lambda i: (0, 0)),
        pl.BlockSpec((8, 24), lambda i: (0, 0)),
    ],
    out_specs=pl.BlockSpec((BLK * IN_DIM // 128, 128), lambda i: (i, 0)),
    out_shape=jax.ShapeDtypeStruct((N_EDGES * IN_DIM // 128, 128),
                                   jnp.float32),
)


@functools.cache
def _make_scatter_kernel():
    mesh = plsc.VectorSubcoreMesh(
        core_axis_name="c", subcore_axis_name="s", num_cores=NC,
        num_subcores=NS)

    @functools.partial(
        pl.kernel,
        out_type=(
            jax.ShapeDtypeStruct((NC, N_PAD, IN_DIM), jnp.float32),
            jax.ShapeDtypeStruct((NC, N_PAD), jnp.float32),
        ),
        mesh=mesh,
        compiler_params=pltpu.CompilerParams(use_tc_tiling_on_sc=False),
        scratch_types=[
            pltpu.VMEM((2, CHUNK), jnp.int32),
            pltpu.VMEM((2, CHUNK, IN_DIM), jnp.float32),
            pltpu.VMEM((CHUNK,), jnp.float32),
            pltpu.VMEM_SHARED((N_PAD, IN_DIM), jnp.float32),
            pltpu.VMEM_SHARED((N_PAD,), jnp.float32),
            pltpu.SemaphoreType.DMA,
            pltpu.SemaphoreType.DMA,
            pltpu.SemaphoreType.DMA,
            pltpu.SemaphoreType.DMA,
        ],
    )
    def _scatter_kernel(rows_hbm, idx_hbm, zeros_hbm, zeros1_hbm, out_hbm,
                        outc_hbm, idx_v, rows_v, ones_v, acc, acc_c,
                        sem_i, sem_r, sem_a, sem_c):
        c = lax.axis_index("c")
        s = lax.axis_index("s")
        wid = s * NC + c
        sl = pl.ds(s * NODES_PER_SUB, NODES_PER_SUB)
        pltpu.sync_copy(zeros_hbm.at[sl], acc.at[sl])
        pltpu.sync_copy(zeros1_hbm.at[sl], acc_c.at[sl])

        def initones(k, carry):
            ones_v[pl.ds(k * 16, 16)] = jnp.ones((16,), jnp.float32)
            return carry

        lax.fori_loop(0, CHUNK // 16, initones, 0)
        plsc.subcore_barrier()

        def src_slice(r):
            return rows_hbm.at[pl.ds((r // 4) * 640, 640),
                               pl.ds((r % 4) * IN_DIM, IN_DIM)]

        n_full = R_GROUPS // NW
        di = pltpu.async_copy(idx_hbm.at[wid], idx_v.at[0], sem_i)
        dr = pltpu.async_copy(src_slice(wid), rows_v.at[0], sem_r)
        da, dc = {}, {}
        for j in range(n_full):
            b = j % 2
            di.wait()
            dr.wait()
            da[j] = pltpu.async_copy(rows_v.at[b], acc.at[idx_v.at[b]],
                                     sem_a, add=True)
            dc[j] = pltpu.async_copy(ones_v, acc_c.at[idx_v.at[b]],
                                     sem_c, add=True)
            if j + 1 < n_full:
                if j >= 1:
                    da[j - 1].wait()
                    dc[j - 1].wait()
                r_next = wid + (j + 1) * NW
                di = pltpu.async_copy(idx_hbm.at[r_next], idx_v.at[1 - b],
                                      sem_i)
                dr = pltpu.async_copy(src_slice(r_next), rows_v.at[1 - b],
                                      sem_r)
        da[n_full - 2].wait()
        dc[n_full - 2].wait()
        da[n_full - 1].wait()
        dc[n_full - 1].wait()
        r_tail = wid + n_full * NW

        @pl.when(r_tail < R_GROUPS)
        def _():
            pltpu.sync_copy(idx_hbm.at[r_tail], idx_v.at[0])
            pltpu.sync_copy(src_slice(r_tail), rows_v.at[0])
            pltpu.sync_copy(rows_v.at[0], acc.at[idx_v.at[0]], add=True)
            pltpu.sync_copy(ones_v, acc_c.at[idx_v.at[0]], add=True)

        plsc.subcore_barrier()
        pltpu.sync_copy(acc.at[sl], out_hbm.at[c, sl])
        pltpu.sync_copy(acc_c.at[sl], outc_hbm.at[c, sl])

    return _scatter_kernel


def _final_body(p_ref, c_ref, bnw_ref, bnb_ref, sv3_ref, r824_ref, out_ref):
    p = p_ref[...]
    sums = (p[0] + p[1])[:N_NODES]
    cnts = c_ref[...][:, :N_NODES]
    cnt = jnp.maximum(cnts[0:1] + cnts[1:2], 1.0).reshape(N_NODES, 1)
    o = sums / cnt
    s = o[:, :MUL]
    v = o[:, MUL:]
    s_mean = jnp.mean(s, axis=0, keepdims=True)
    s_c = s - s_mean
    s_var = jnp.mean(s_c * s_c, axis=0, keepdims=True)
    bnw = bnw_ref[...]
    s_out = s_c * (lax.rsqrt(s_var + EPS) * bnw[:, :MUL]) + bnb_ref[...]
    vsq = jnp.mean(v * v, axis=0, keepdims=True)
    v_norm = jnp.dot(vsq, sv3_ref[...], preferred_element_type=jnp.float32)
    vfac = lax.rsqrt(v_norm + EPS) * bnw[:, MUL:]
    vfac24 = jnp.dot(vfac, r824_ref[...], preferred_element_type=jnp.float32)
    out_ref[...] = jnp.concatenate([s_out, v * vfac24], axis=1)


_final_call = pl.pallas_call(
    _final_body,
    in_specs=[
        pl.BlockSpec((NC, N_PAD, IN_DIM), lambda: (0, 0, 0)),
        pl.BlockSpec((NC, N_PAD), lambda: (0, 0)),
        pl.BlockSpec((1, 2 * MUL), lambda: (0, 0)),
        pl.BlockSpec((1, MUL), lambda: (0, 0)),
        pl.BlockSpec((24, 8), lambda: (0, 0)),
        pl.BlockSpec((8, 24), lambda: (0, 0)),
    ],
    out_specs=pl.BlockSpec((N_NODES, IN_DIM), lambda: (0, 0)),
    out_shape=jax.ShapeDtypeStruct((N_NODES, IN_DIM), jnp.float32),
)


def kernel(src_features, edge_sh, edge_scalars, edge_index, W1, b1, W2, b2,
           bn_weight, bn_bias):
    src2d = edge_index[0].reshape(R_GROUPS, CHUNK)
    dst2d = edge_index[1].reshape(R_GROUPS, CHUNK)
    win = (
        jnp.zeros((36, 56), jnp.float32)
        .at[:SCAL_DIM, :HID].set(W1)
        .at[SCAL_DIM:, HID:].set(_PRE_NP)
        .T.astype(jnp.bfloat16)
    )
    b_in = jnp.concatenate([b1, jnp.zeros((24,), jnp.float32)]).reshape(56, 1)
    w2f = (W2 @ _F_NP).T.astype(jnp.bfloat16)
    b2f = (b2 @ _F_NP).reshape(384, 1).astype(jnp.bfloat16)
    xg128 = _make_gather_kernel()(src_features, src2d)
    out_tp = _edge_call(edge_scalars.T, xg128, edge_sh.T, win, b_in, w2f, b2f,
                        _EBIG_NP.T.astype(jnp.bfloat16),
                        _RBIG_NP.T.astype(jnp.bfloat16),
                        _SV1_NP.astype(jnp.bfloat16))
    partial, cnts = _make_scatter_kernel()(
        out_tp, dst2d, jnp.zeros((N_PAD, IN_DIM), jnp.float32),
        jnp.zeros((N_PAD,), jnp.float32))
    return _final_call(partial, cnts, bn_weight.reshape(1, 2 * MUL),
                       bn_bias.reshape(1, MUL), _SV3_NP, _R824_NP)
